# baseline (speedup 1.0000x reference)
import functools

import jax
import jax.numpy as jnp
from jax import lax
from jax.experimental import pallas as pl
from jax.experimental.pallas import tpu as pltpu
from jax.experimental.pallas import tpu_sc as plsc

N = 10000
E = 320000
NFEAT = 128
NHID = 128
NCLASS = 16
EPS = 1e-5

NC = 2
NS = 16
NW = NC * NS
EW = E // NW
B = 100
KS = EW // B
B16 = 125
KS16 = EW // B16
B50 = 25
KS50 = EW // B50
RPT = 640
NP = RPT * NS

R = 1000
GRID = N // R


def _sc_agg(feat, b, ks, nbuf):
    assert ks % nbuf == 0
    mesh = plsc.VectorSubcoreMesh(core_axis_name="c", subcore_axis_name="s")

    @functools.partial(
        pl.kernel,
        out_type=jax.ShapeDtypeStruct((NC, NP, feat), jnp.float32),
        mesh=mesh,
        scratch_types=[
            pltpu.VMEM((ks, b), jnp.int32),
            pltpu.VMEM((ks, b), jnp.int32),
            [pltpu.VMEM((b, feat), jnp.float32)] * nbuf,
            pltpu.VMEM_SHARED((NP, feat), jnp.float32),
            [pltpu.SemaphoreType.DMA] * nbuf,
            [pltpu.SemaphoreType.DMA] * nbuf,
        ],
        compiler_params=pltpu.CompilerParams(use_tc_tiling_on_sc=False),
    )
    def agg(g_hbm, src_hbm, dst_hbm, zrow_hbm, out_hbm,
            sidx, didx, rows, acc, gsem, ssem):
        c = lax.axis_index("c")
        s = lax.axis_index("s")
        wid = s * NC + c
        pltpu.sync_copy(src_hbm.at[wid], sidx)
        pltpu.sync_copy(dst_hbm.at[wid], didx)
        for j in range(nbuf - 1):
            pltpu.async_copy(g_hbm.at[sidx.at[j]], rows[j], gsem[j])
        pltpu.sync_copy(zrow_hbm, acc.at[pl.ds(s * RPT, RPT)])
        plsc.subcore_barrier()

        def outer(i, carry):
            for bb in range(nbuf):
                j = i * nbuf + bb
                jn = j + nbuf - 1
                bn = (bb + nbuf - 1) % nbuf

                @pl.when(jn < ks)
                def _issue():
                    def drain_then_gather():
                        pltpu.make_async_copy(
                            rows[bn], acc.at[didx.at[jn - nbuf]], ssem[bn]).wait()

                    if bb > 0:
                        drain_then_gather()
                    else:
                        pl.when(i >= 1)(drain_then_gather)
                    pltpu.async_copy(g_hbm.at[sidx.at[jn]], rows[bn], gsem[bn])

                pltpu.make_async_copy(g_hbm.at[sidx.at[j]], rows[bb], gsem[bb]).wait()
                pltpu.async_copy(rows[bb], acc.at[didx.at[j]], ssem[bb], add=True)
            return carry

        lax.fori_loop(0, ks // nbuf, outer, 0)
        for bb in range(nbuf):
            pltpu.make_async_copy(
                rows[bb], acc.at[didx.at[ks - nbuf + bb]], ssem[bb]).wait()
        plsc.subcore_barrier()
        pltpu.sync_copy(acc.at[pl.ds(s * RPT, RPT)],
                        out_hbm.at[c, pl.ds(s * RPT, RPT)])

    return agg


def _sc_deg():
    mesh = plsc.VectorSubcoreMesh(core_axis_name="c", subcore_axis_name="s")

    @functools.partial(
        pl.kernel,
        out_type=jax.ShapeDtypeStruct((NC, NP, NCLASS), jnp.float32),
        mesh=mesh,
        scratch_types=[
            pltpu.VMEM((KS16, B16), jnp.int32),
            pltpu.VMEM((B16, NCLASS), jnp.float32),
            pltpu.VMEM_SHARED((NP, NCLASS), jnp.float32),
        ],
        compiler_params=pltpu.CompilerParams(use_tc_tiling_on_sc=False),
    )
    def deg(ones_hbm, dst_hbm, zrow_hbm, out_hbm, didx, ones_v, acc):
        c = lax.axis_index("c")
        s = lax.axis_index("s")
        wid = s * NC + c
        pltpu.sync_copy(zrow_hbm, acc.at[pl.ds(s * RPT, RPT)])
        pltpu.sync_copy(dst_hbm.at[wid], didx)
        pltpu.sync_copy(ones_hbm, ones_v)
        plsc.subcore_barrier()

        def step(j, carry):
            pltpu.sync_copy(ones_v, acc.at[didx.at[j]], add=True)
            return carry

        lax.fori_loop(0, KS16, step, 0)
        plsc.subcore_barrier()
        pltpu.sync_copy(acc.at[pl.ds(s * RPT, RPT)],
                        out_hbm.at[c, pl.ds(s * RPT, RPT)])

    return deg


def _t_first(degp, x, w1):
    def body(degp_ref, x_ref, w_ref, dinv_ref, g_ref):
        deg = degp_ref[0, :, 0:1] + degp_ref[1, :, 0:1] + 1.0
        dinv = lax.rsqrt(deg)
        dinv_ref[...] = dinv
        g_ref[...] = jnp.dot(x_ref[...], w_ref[...],
                             preferred_element_type=jnp.float32) * dinv

    return pl.pallas_call(
        body,
        grid=(GRID,),
        in_specs=[
            pl.BlockSpec((NC, R, NCLASS), lambda i: (0, i, 0)),
            pl.BlockSpec((R, NFEAT), lambda i: (i, 0)),
            pl.BlockSpec((NFEAT, NHID), lambda i: (0, 0)),
        ],
        out_specs=[
            pl.BlockSpec((R, 1), lambda i: (i, 0)),
            pl.BlockSpec((R, NHID), lambda i: (i, 0)),
        ],
        out_shape=[
            jax.ShapeDtypeStruct((N, 1), jnp.float32),
            jax.ShapeDtypeStruct((N, NHID), jnp.float32),
        ],
    )(degp, x, w1)


def _t_mid(parts, g, dinv, b, lng, lnb, w):
    fout = w.shape[1]

    def body(p_ref, g_ref, dinv_ref, b_ref, lng_ref, lnb_ref, w_ref, out_ref):
        dinv = dinv_ref[...]
        agg = (p_ref[0] + p_ref[1] + g_ref[...]) * dinv + b_ref[...]
        mean = jnp.mean(agg, axis=-1, keepdims=True)
        cen = agg - mean
        var = jnp.mean(cen * cen, axis=-1, keepdims=True)
        xn = cen * lax.rsqrt(var + EPS) * lng_ref[...] + lnb_ref[...]
        xr = jnp.maximum(xn, 0.0)
        out_ref[...] = jnp.dot(xr, w_ref[...],
                               preferred_element_type=jnp.float32) * dinv

    return pl.pallas_call(
        body,
        grid=(GRID,),
        in_specs=[
            pl.BlockSpec((NC, R, NHID), lambda i: (0, i, 0)),
            pl.BlockSpec((R, NHID), lambda i: (i, 0)),
            pl.BlockSpec((R, 1), lambda i: (i, 0)),
            pl.BlockSpec((1, NHID), lambda i: (0, 0)),
            pl.BlockSpec((1, NHID), lambda i: (0, 0)),
            pl.BlockSpec((1, NHID), lambda i: (0, 0)),
            pl.BlockSpec((NHID, fout), lambda i: (0, 0)),
        ],
        out_specs=pl.BlockSpec((R, fout), lambda i: (i, 0)),
        out_shape=jax.ShapeDtypeStruct((N, fout), jnp.float32),
    )(parts, g, dinv, b, lng, lnb, w)


def _t_last(parts, g, dinv, b):
    def body(p_ref, g_ref, dinv_ref, b_ref, out_ref):
        logits = (p_ref[0] + p_ref[1] + g_ref[...]) * dinv_ref[...] + b_ref[...]
        m = jnp.max(logits, axis=-1, keepdims=True)
        z = logits - m
        lse = jnp.log(jnp.sum(jnp.exp(z), axis=-1, keepdims=True))
        out_ref[...] = z - lse

    return pl.pallas_call(
        body,
        grid=(GRID,),
        in_specs=[
            pl.BlockSpec((NC, R, NCLASS), lambda i: (0, i, 0)),
            pl.BlockSpec((R, NCLASS), lambda i: (i, 0)),
            pl.BlockSpec((R, 1), lambda i: (i, 0)),
            pl.BlockSpec((1, NCLASS), lambda i: (0, 0)),
        ],
        out_specs=pl.BlockSpec((R, NCLASS), lambda i: (i, 0)),
        out_shape=jax.ShapeDtypeStruct((N, NCLASS), jnp.float32),
    )(parts, g, dinv, b)


def kernel(x, edge_index, W1, b1, W2, b2, W3, b3, ln1_g, ln1_b, ln2_g, ln2_b):
    src3d = edge_index[0].reshape(NW, KS, B)
    dst3d = edge_index[1].reshape(NW, KS, B)
    src3d16 = edge_index[0].reshape(NW, KS16, B16)
    dst3d16 = edge_index[1].reshape(NW, KS16, B16)
    src50 = edge_index[0].reshape(NW, KS50, B50)
    dst50 = edge_index[1].reshape(NW, KS50, B50)
    ones16 = jnp.ones((B16, NCLASS), jnp.float32)
    z16 = jnp.zeros((RPT, NCLASS), jnp.float32)
    z128 = jnp.zeros((RPT, NHID), jnp.float32)
    b1r, b2r = b1.reshape(1, NHID), b2.reshape(1, NHID)
    b3r = b3.reshape(1, NCLASS)
    ln1gr, ln1br = ln1_g.reshape(1, NHID), ln1_b.reshape(1, NHID)
    ln2gr, ln2br = ln2_g.reshape(1, NHID), ln2_b.reshape(1, NHID)

    degp = _sc_deg()(ones16, dst3d16, z16)
    dinv, g1 = _t_first(degp, x, W1)
    p1 = _sc_agg(NHID, B50, KS50, 5)(g1, src50, dst50, z128)
    g2 = _t_mid(p1, g1, dinv, b1r, ln1gr, ln1br, W2)
    p2 = _sc_agg(NHID, B50, KS50, 5)(g2, src50, dst50, z128)
    g3 = _t_mid(p2, g2, dinv, b2r, ln2gr, ln2br, W3)
    p3 = _sc_agg(NCLASS, B16, KS16, 8)(g3, src3d16, dst3d16, z16)
    return _t_last(p3, g3, dinv, b3r)


# speedup vs baseline: 1.1485x; 1.1485x over previous
import functools

import jax
import jax.numpy as jnp
from jax import lax
from jax.experimental import pallas as pl
from jax.experimental.pallas import tpu as pltpu
from jax.experimental.pallas import tpu_sc as plsc

N = 10000
E = 320000
NFEAT = 128
NHID = 128
NCLASS = 16
EPS = 1e-5

NC = 2
NS = 16
NW = NC * NS
EW = E // NW
B = 100
KS = EW // B
B16 = 125
KS16 = EW // B16
B50 = 40
KS50 = EW // B50
RPT = 640
NP = RPT * NS

R = 1000
GRID = N // R


def _sc_agg(feat, b, ks, nbuf):
    assert ks % nbuf == 0
    mesh = plsc.VectorSubcoreMesh(core_axis_name="c", subcore_axis_name="s")

    @functools.partial(
        pl.kernel,
        out_type=jax.ShapeDtypeStruct((NC, NP, feat), jnp.float32),
        mesh=mesh,
        scratch_types=[
            pltpu.VMEM((ks, b), jnp.int32),
            pltpu.VMEM((ks, b), jnp.int32),
            [pltpu.VMEM((b, feat), jnp.float32)] * nbuf,
            pltpu.VMEM_SHARED((NP, feat), jnp.float32),
            [pltpu.SemaphoreType.DMA] * nbuf,
            [pltpu.SemaphoreType.DMA] * nbuf,
        ],
        compiler_params=pltpu.CompilerParams(use_tc_tiling_on_sc=False),
    )
    def agg(g_hbm, src_hbm, dst_hbm, zrow_hbm, out_hbm,
            sidx, didx, rows, acc, gsem, ssem):
        c = lax.axis_index("c")
        s = lax.axis_index("s")
        wid = s * NC + c
        pltpu.sync_copy(src_hbm.at[wid], sidx)
        pltpu.sync_copy(dst_hbm.at[wid], didx)
        for j in range(nbuf - 1):
            pltpu.async_copy(g_hbm.at[sidx.at[j]], rows[j], gsem[j])
        pltpu.sync_copy(zrow_hbm, acc.at[pl.ds(s * RPT, RPT)])
        plsc.subcore_barrier()

        def outer(i, carry):
            for bb in range(nbuf):
                j = i * nbuf + bb
                jn = j + nbuf - 1
                bn = (bb + nbuf - 1) % nbuf

                @pl.when(jn < ks)
                def _issue():
                    def drain_then_gather():
                        pltpu.make_async_copy(
                            rows[bn], acc.at[didx.at[jn - nbuf]], ssem[bn]).wait()

                    if bb > 0:
                        drain_then_gather()
                    else:
                        pl.when(i >= 1)(drain_then_gather)
                    pltpu.async_copy(g_hbm.at[sidx.at[jn]], rows[bn], gsem[bn])

                pltpu.make_async_copy(g_hbm.at[sidx.at[j]], rows[bb], gsem[bb]).wait()
                pltpu.async_copy(rows[bb], acc.at[didx.at[j]], ssem[bb], add=True)
            return carry

        lax.fori_loop(0, ks // nbuf, outer, 0)
        for bb in range(nbuf):
            pltpu.make_async_copy(
                rows[bb], acc.at[didx.at[ks - nbuf + bb]], ssem[bb]).wait()
        plsc.subcore_barrier()
        pltpu.sync_copy(acc.at[pl.ds(s * RPT, RPT)],
                        out_hbm.at[c, pl.ds(s * RPT, RPT)])

    return agg


def _sc_deg():
    mesh = plsc.VectorSubcoreMesh(core_axis_name="c", subcore_axis_name="s")

    @functools.partial(
        pl.kernel,
        out_type=jax.ShapeDtypeStruct((NC, NP, NCLASS), jnp.float32),
        mesh=mesh,
        scratch_types=[
            pltpu.VMEM((KS16, B16), jnp.int32),
            pltpu.VMEM((B16, NCLASS), jnp.float32),
            pltpu.VMEM_SHARED((NP, NCLASS), jnp.float32),
        ],
        compiler_params=pltpu.CompilerParams(use_tc_tiling_on_sc=False),
    )
    def deg(ones_hbm, dst_hbm, zrow_hbm, out_hbm, didx, ones_v, acc):
        c = lax.axis_index("c")
        s = lax.axis_index("s")
        wid = s * NC + c
        pltpu.sync_copy(zrow_hbm, acc.at[pl.ds(s * RPT, RPT)])
        pltpu.sync_copy(dst_hbm.at[wid], didx)
        pltpu.sync_copy(ones_hbm, ones_v)
        plsc.subcore_barrier()

        def step(j, carry):
            pltpu.sync_copy(ones_v, acc.at[didx.at[j]], add=True)
            return carry

        lax.fori_loop(0, KS16, step, 0)
        plsc.subcore_barrier()
        pltpu.sync_copy(acc.at[pl.ds(s * RPT, RPT)],
                        out_hbm.at[c, pl.ds(s * RPT, RPT)])

    return deg


def _t_first(degp, x, w1):
    def body(degp_ref, x_ref, w_ref, dinv_ref, g_ref):
        deg = degp_ref[0, :, 0:1] + degp_ref[1, :, 0:1] + 1.0
        dinv = lax.rsqrt(deg)
        dinv_ref[...] = dinv
        g_ref[...] = jnp.dot(x_ref[...], w_ref[...],
                             preferred_element_type=jnp.float32) * dinv

    return pl.pallas_call(
        body,
        grid=(GRID,),
        in_specs=[
            pl.BlockSpec((NC, R, NCLASS), lambda i: (0, i, 0)),
            pl.BlockSpec((R, NFEAT), lambda i: (i, 0)),
            pl.BlockSpec((NFEAT, NHID), lambda i: (0, 0)),
        ],
        out_specs=[
            pl.BlockSpec((R, 1), lambda i: (i, 0)),
            pl.BlockSpec((R, NHID), lambda i: (i, 0)),
        ],
        out_shape=[
            jax.ShapeDtypeStruct((N, 1), jnp.float32),
            jax.ShapeDtypeStruct((N, NHID), jnp.float32),
        ],
    )(degp, x, w1)


def _t_mid(parts, g, dinv, b, lng, lnb, w):
    fout = w.shape[1]

    def body(p_ref, g_ref, dinv_ref, b_ref, lng_ref, lnb_ref, w_ref, out_ref):
        dinv = dinv_ref[...]
        agg = (p_ref[0] + p_ref[1] + g_ref[...]) * dinv + b_ref[...]
        mean = jnp.mean(agg, axis=-1, keepdims=True)
        cen = agg - mean
        var = jnp.mean(cen * cen, axis=-1, keepdims=True)
        xn = cen * lax.rsqrt(var + EPS) * lng_ref[...] + lnb_ref[...]
        xr = jnp.maximum(xn, 0.0)
        out_ref[...] = jnp.dot(xr, w_ref[...],
                               preferred_element_type=jnp.float32) * dinv

    return pl.pallas_call(
        body,
        grid=(GRID,),
        in_specs=[
            pl.BlockSpec((NC, R, NHID), lambda i: (0, i, 0)),
            pl.BlockSpec((R, NHID), lambda i: (i, 0)),
            pl.BlockSpec((R, 1), lambda i: (i, 0)),
            pl.BlockSpec((1, NHID), lambda i: (0, 0)),
            pl.BlockSpec((1, NHID), lambda i: (0, 0)),
            pl.BlockSpec((1, NHID), lambda i: (0, 0)),
            pl.BlockSpec((NHID, fout), lambda i: (0, 0)),
        ],
        out_specs=pl.BlockSpec((R, fout), lambda i: (i, 0)),
        out_shape=jax.ShapeDtypeStruct((N, fout), jnp.float32),
    )(parts, g, dinv, b, lng, lnb, w)


def _t_last(parts, g, dinv, b):
    def body(p_ref, g_ref, dinv_ref, b_ref, out_ref):
        logits = (p_ref[0] + p_ref[1] + g_ref[...]) * dinv_ref[...] + b_ref[...]
        m = jnp.max(logits, axis=-1, keepdims=True)
        z = logits - m
        lse = jnp.log(jnp.sum(jnp.exp(z), axis=-1, keepdims=True))
        out_ref[...] = z - lse

    return pl.pallas_call(
        body,
        grid=(GRID,),
        in_specs=[
            pl.BlockSpec((NC, R, NCLASS), lambda i: (0, i, 0)),
            pl.BlockSpec((R, NCLASS), lambda i: (i, 0)),
            pl.BlockSpec((R, 1), lambda i: (i, 0)),
            pl.BlockSpec((1, NCLASS), lambda i: (0, 0)),
        ],
        out_specs=pl.BlockSpec((R, NCLASS), lambda i: (i, 0)),
        out_shape=jax.ShapeDtypeStruct((N, NCLASS), jnp.float32),
    )(parts, g, dinv, b)


def kernel(x, edge_index, W1, b1, W2, b2, W3, b3, ln1_g, ln1_b, ln2_g, ln2_b):
    src3d = edge_index[0].reshape(NW, KS, B)
    dst3d = edge_index[1].reshape(NW, KS, B)
    src3d16 = edge_index[0].reshape(NW, KS16, B16)
    dst3d16 = edge_index[1].reshape(NW, KS16, B16)
    src50 = edge_index[0].reshape(NW, KS50, B50)
    dst50 = edge_index[1].reshape(NW, KS50, B50)
    ones16 = jnp.ones((B16, NCLASS), jnp.float32)
    z16 = jnp.zeros((RPT, NCLASS), jnp.float32)
    z128 = jnp.zeros((RPT, NHID), jnp.float32)
    b1r, b2r = b1.reshape(1, NHID), b2.reshape(1, NHID)
    b3r = b3.reshape(1, NCLASS)
    ln1gr, ln1br = ln1_g.reshape(1, NHID), ln1_b.reshape(1, NHID)
    ln2gr, ln2br = ln2_g.reshape(1, NHID), ln2_b.reshape(1, NHID)

    degp = _sc_deg()(ones16, dst3d16, z16)
    dinv, g1 = _t_first(degp, x, W1)
    p1 = _sc_agg(NHID, B50, KS50, 5)(g1, src50, dst50, z128)
    g2 = _t_mid(p1, g1, dinv, b1r, ln1gr, ln1br, W2)
    p2 = _sc_agg(NHID, B50, KS50, 5)(g2, src50, dst50, z128)
    g3 = _t_mid(p2, g2, dinv, b2r, ln2gr, ln2br, W3)
    p3 = _sc_agg(NCLASS, B16, KS16, 8)(g3, src3d16, dst3d16, z16)
    return _t_last(p3, g3, dinv, b3r)


# speedup vs baseline: 1.1486x; 1.0001x over previous
import functools

import jax
import jax.numpy as jnp
from jax import lax
from jax.experimental import pallas as pl
from jax.experimental.pallas import tpu as pltpu
from jax.experimental.pallas import tpu_sc as plsc

N = 10000
E = 320000
NFEAT = 128
NHID = 128
NCLASS = 16
EPS = 1e-5

NC = 2
NS = 16
NW = NC * NS
EW = E // NW
B16 = 125
KS16 = EW // B16
B50 = 40
KS50 = EW // B50
RPT = 640
NP = RPT * NS

R = 1000
GRID = N // R


def _sc_agg(feat, b, ks, nbuf):
    assert ks % nbuf == 0
    mesh = plsc.VectorSubcoreMesh(core_axis_name="c", subcore_axis_name="s")

    @functools.partial(
        pl.kernel,
        out_type=jax.ShapeDtypeStruct((NC, NP, feat), jnp.float32),
        mesh=mesh,
        scratch_types=[
            pltpu.VMEM((ks, b), jnp.int32),
            pltpu.VMEM((ks, b), jnp.int32),
            [pltpu.VMEM((b, feat), jnp.float32)] * nbuf,
            pltpu.VMEM_SHARED((NP, feat), jnp.float32),
            [pltpu.SemaphoreType.DMA] * nbuf,
            [pltpu.SemaphoreType.DMA] * nbuf,
        ],
        compiler_params=pltpu.CompilerParams(use_tc_tiling_on_sc=False),
    )
    def agg(g_hbm, src_hbm, dst_hbm, zrow_hbm, out_hbm,
            sidx, didx, rows, acc, gsem, ssem):
        c = lax.axis_index("c")
        s = lax.axis_index("s")
        wid = s * NC + c
        pltpu.sync_copy(src_hbm.at[wid], sidx)
        pltpu.sync_copy(dst_hbm.at[wid], didx)
        for j in range(nbuf - 1):
            pltpu.async_copy(g_hbm.at[sidx.at[j]], rows[j], gsem[j])
        pltpu.sync_copy(zrow_hbm, acc.at[pl.ds(s * RPT, RPT)])
        plsc.subcore_barrier()

        def outer(i, carry):
            for bb in range(nbuf):
                j = i * nbuf + bb
                jn = j + nbuf - 1
                bn = (bb + nbuf - 1) % nbuf

                @pl.when(jn < ks)
                def _issue():
                    def drain_then_gather():
                        pltpu.make_async_copy(
                            rows[bn], acc.at[didx.at[jn - nbuf]], ssem[bn]).wait()

                    if bb > 0:
                        drain_then_gather()
                    else:
                        pl.when(i >= 1)(drain_then_gather)
                    pltpu.async_copy(g_hbm.at[sidx.at[jn]], rows[bn], gsem[bn])

                pltpu.make_async_copy(g_hbm.at[sidx.at[j]], rows[bb], gsem[bb]).wait()
                pltpu.async_copy(rows[bb], acc.at[didx.at[j]], ssem[bb], add=True)
            return carry

        lax.fori_loop(0, ks // nbuf, outer, 0)
        for bb in range(nbuf):
            pltpu.make_async_copy(
                rows[bb], acc.at[didx.at[ks - nbuf + bb]], ssem[bb]).wait()
        plsc.subcore_barrier()
        pltpu.sync_copy(acc.at[pl.ds(s * RPT, RPT)],
                        out_hbm.at[c, pl.ds(s * RPT, RPT)])

    return agg


def _sc_deg():
    mesh = plsc.VectorSubcoreMesh(core_axis_name="c", subcore_axis_name="s")

    @functools.partial(
        pl.kernel,
        out_type=jax.ShapeDtypeStruct((NC, NP, NCLASS), jnp.float32),
        mesh=mesh,
        scratch_types=[
            pltpu.VMEM((KS16, B16), jnp.int32),
            pltpu.VMEM((B16, NCLASS), jnp.float32),
            pltpu.VMEM_SHARED((NP, NCLASS), jnp.float32),
        ],
        compiler_params=pltpu.CompilerParams(use_tc_tiling_on_sc=False),
    )
    def deg(ones_hbm, dst_hbm, zrow_hbm, out_hbm, didx, ones_v, acc):
        c = lax.axis_index("c")
        s = lax.axis_index("s")
        wid = s * NC + c
        pltpu.sync_copy(zrow_hbm, acc.at[pl.ds(s * RPT, RPT)])
        pltpu.sync_copy(dst_hbm.at[wid], didx)
        pltpu.sync_copy(ones_hbm, ones_v)
        plsc.subcore_barrier()

        def step(j, carry):
            pltpu.sync_copy(ones_v, acc.at[didx.at[j]], add=True)
            return carry

        lax.fori_loop(0, KS16, step, 0)
        plsc.subcore_barrier()
        pltpu.sync_copy(acc.at[pl.ds(s * RPT, RPT)],
                        out_hbm.at[c, pl.ds(s * RPT, RPT)])

    return deg


def _t_first(degp, x, w1):
    def body(degp_ref, x_ref, w_ref, dinv_ref, g_ref):
        deg = degp_ref[0, :, 0:1] + degp_ref[1, :, 0:1] + 1.0
        dinv = lax.rsqrt(deg)
        dinv_ref[...] = dinv
        g_ref[...] = jnp.dot(x_ref[...], w_ref[...],
                             preferred_element_type=jnp.float32) * dinv

    return pl.pallas_call(
        body,
        grid=(GRID,),
        in_specs=[
            pl.BlockSpec((NC, R, NCLASS), lambda i: (0, i, 0)),
            pl.BlockSpec((R, NFEAT), lambda i: (i, 0)),
            pl.BlockSpec((NFEAT, NHID), lambda i: (0, 0)),
        ],
        out_specs=[
            pl.BlockSpec((R, 1), lambda i: (i, 0)),
            pl.BlockSpec((R, NHID), lambda i: (i, 0)),
        ],
        out_shape=[
            jax.ShapeDtypeStruct((N, 1), jnp.float32),
            jax.ShapeDtypeStruct((N, NHID), jnp.float32),
        ],
    )(degp, x, w1)


def _t_mid(parts, g, dinv, b, lng, lnb, w):
    fout = w.shape[1]

    def body(p_ref, g_ref, dinv_ref, b_ref, lng_ref, lnb_ref, w_ref, out_ref):
        dinv = dinv_ref[...]
        agg = (p_ref[0] + p_ref[1] + g_ref[...]) * dinv + b_ref[...]
        mean = jnp.mean(agg, axis=-1, keepdims=True)
        cen = agg - mean
        var = jnp.mean(cen * cen, axis=-1, keepdims=True)
        xn = cen * lax.rsqrt(var + EPS) * lng_ref[...] + lnb_ref[...]
        xr = jnp.maximum(xn, 0.0)
        out_ref[...] = jnp.dot(xr, w_ref[...],
                               preferred_element_type=jnp.float32) * dinv

    return pl.pallas_call(
        body,
        grid=(GRID,),
        in_specs=[
            pl.BlockSpec((NC, R, NHID), lambda i: (0, i, 0)),
            pl.BlockSpec((R, NHID), lambda i: (i, 0)),
            pl.BlockSpec((R, 1), lambda i: (i, 0)),
            pl.BlockSpec((1, NHID), lambda i: (0, 0)),
            pl.BlockSpec((1, NHID), lambda i: (0, 0)),
            pl.BlockSpec((1, NHID), lambda i: (0, 0)),
            pl.BlockSpec((NHID, fout), lambda i: (0, 0)),
        ],
        out_specs=pl.BlockSpec((R, fout), lambda i: (i, 0)),
        out_shape=jax.ShapeDtypeStruct((N, fout), jnp.float32),
    )(parts, g, dinv, b, lng, lnb, w)


def _t_last(parts, g, dinv, b):
    def body(p_ref, g_ref, dinv_ref, b_ref, out_ref):
        logits = (p_ref[0] + p_ref[1] + g_ref[...]) * dinv_ref[...] + b_ref[...]
        m = jnp.max(logits, axis=-1, keepdims=True)
        z = logits - m
        lse = jnp.log(jnp.sum(jnp.exp(z), axis=-1, keepdims=True))
        out_ref[...] = z - lse

    return pl.pallas_call(
        body,
        grid=(GRID,),
        in_specs=[
            pl.BlockSpec((NC, R, NCLASS), lambda i: (0, i, 0)),
            pl.BlockSpec((R, NCLASS), lambda i: (i, 0)),
            pl.BlockSpec((R, 1), lambda i: (i, 0)),
            pl.BlockSpec((1, NCLASS), lambda i: (0, 0)),
        ],
        out_specs=pl.BlockSpec((R, NCLASS), lambda i: (i, 0)),
        out_shape=jax.ShapeDtypeStruct((N, NCLASS), jnp.float32),
    )(parts, g, dinv, b)


def kernel(x, edge_index, W1, b1, W2, b2, W3, b3, ln1_g, ln1_b, ln2_g, ln2_b):
    src3d16 = edge_index[0].reshape(NW, KS16, B16)
    dst3d16 = edge_index[1].reshape(NW, KS16, B16)
    src50 = edge_index[0].reshape(NW, KS50, B50)
    dst50 = edge_index[1].reshape(NW, KS50, B50)
    ones16 = jnp.ones((B16, NCLASS), jnp.float32)
    z16 = jnp.zeros((RPT, NCLASS), jnp.float32)
    z128 = jnp.zeros((RPT, NHID), jnp.float32)
    b1r, b2r = b1.reshape(1, NHID), b2.reshape(1, NHID)
    b3r = b3.reshape(1, NCLASS)
    ln1gr, ln1br = ln1_g.reshape(1, NHID), ln1_b.reshape(1, NHID)
    ln2gr, ln2br = ln2_g.reshape(1, NHID), ln2_b.reshape(1, NHID)

    degp = _sc_deg()(ones16, dst3d16, z16)
    dinv, g1 = _t_first(degp, x, W1)
    p1 = _sc_agg(NHID, B50, KS50, 5)(g1, src50, dst50, z128)
    g2 = _t_mid(p1, g1, dinv, b1r, ln1gr, ln1br, W2)
    p2 = _sc_agg(NHID, B50, KS50, 5)(g2, src50, dst50, z128)
    g3 = _t_mid(p2, g2, dinv, b2r, ln2gr, ln2br, W3)
    p3 = _sc_agg(NCLASS, B16, KS16, 8)(g3, src3d16, dst3d16, z16)
    return _t_last(p3, g3, dinv, b3r)
